# baseline (speedup 1.0000x reference)
import functools

import jax
import jax.numpy as jnp
from jax import lax
from jax.experimental import pallas as pl
from jax.experimental.pallas import tpu as pltpu
from jax.experimental.pallas import tpu_sc as plsc

N = 10000
NP = 10240
D = 128
E = 320000
NC = 2
NS = 16
NW = NC * NS
CH = 64
EPW = E // NW
CPW = 160
NG = 5
CPG = CPW // NG
RPS = NP // NS


def _sc_aggregate(h, src2d, dst2d, zeros):
    mesh = plsc.VectorSubcoreMesh(core_axis_name="c", subcore_axis_name="s")

    @functools.partial(
        pl.kernel,
        out_type=(jax.ShapeDtypeStruct((NP, D), jnp.float32),
                  jax.ShapeDtypeStruct((NP, D), jnp.float32)),
        mesh=mesh,
        scratch_types=[
            pltpu.VMEM((CPG, CH), jnp.int32),
            pltpu.VMEM((CPG, CH), jnp.int32),
            pltpu.VMEM((CPG, CH), jnp.int32),
            pltpu.VMEM((CPG, CH), jnp.int32),
            pltpu.VMEM((CH, D), jnp.float32),
            pltpu.VMEM((CH, D), jnp.float32),
            pltpu.VMEM((CH, D), jnp.float32),
            pltpu.VMEM((CH, D), jnp.float32),
            pltpu.VMEM_SHARED((NP, D), jnp.float32),
            pltpu.SemaphoreType.DMA,
            pltpu.SemaphoreType.DMA,
            pltpu.SemaphoreType.DMA,
            pltpu.SemaphoreType.DMA,
            pltpu.SemaphoreType.DMA,
            pltpu.SemaphoreType.DMA,
            pltpu.SemaphoreType.DMA,
            pltpu.SemaphoreType.DMA,
            pltpu.SemaphoreType.DMA,
        ],
    )
    def k(h_hbm, src_hbm, dst_hbm, zeros_hbm, p0_hbm, p1_hbm,
          srcv0, srcv1, dstv0, dstv1, b0, b1, b2, b3, acc,
          g0, g1, g2, g3, s0, s1, s2, s3, isem):
        c = lax.axis_index("c")
        s = lax.axis_index("s")
        row0 = s * RPS

        @pl.when(c == 0)
        def _():
            pltpu.sync_copy(h_hbm.at[pl.ds(row0, RPS)],
                            acc.at[pl.ds(row0, RPS)])

        @pl.when(c == 1)
        def _():
            pltpu.sync_copy(zeros_hbm.at[pl.ds(row0, RPS)],
                            acc.at[pl.ds(row0, RPS)])

        w = c * NS + s
        srcv = (srcv0, srcv1)
        dstv = (dstv0, dstv1)
        pltpu.sync_copy(src_hbm.at[w, 0], srcv0)
        pltpu.sync_copy(dst_hbm.at[w, 0], dstv0)

        plsc.subcore_barrier()

        bufs = (b0, b1, b2, b3)
        gsem = (g0, g1, g2, g3)
        ssem = (s0, s1, s2, s3)

        for g in range(NG):
            sv = srcv[g % 2]
            dv = dstv[g % 2]
            if g + 1 < NG:
                pltpu.async_copy(src_hbm.at[w, g + 1], srcv[(g + 1) % 2], isem)
                pltpu.async_copy(dst_hbm.at[w, g + 1], dstv[(g + 1) % 2], isem)

            pltpu.async_copy(h_hbm.at[sv.at[0]], bufs[0], gsem[0])
            pltpu.async_copy(h_hbm.at[sv.at[1]], bufs[1], gsem[1])

            @pl.loop(0, CPG, step=4)
            def _(j, sv=sv, dv=dv):
                for b in range(4):
                    c = j + b
                    b2 = (b + 2) % 4
                    pltpu.make_async_copy(h_hbm.at[sv.at[c]], bufs[b],
                                          gsem[b]).wait()
                    pltpu.async_copy(bufs[b], acc.at[dv.at[c]], ssem[b],
                                     add=True)

                    @pl.when((c >= 2) & (c + 2 < CPG))
                    def _(b2=b2, c=c):
                        pltpu.make_async_copy(bufs[b2], acc.at[dv.at[c - 2]],
                                              ssem[b2]).wait()

                    @pl.when(c + 2 < CPG)
                    def _(b2=b2, c=c):
                        pltpu.async_copy(h_hbm.at[sv.at[c + 2]], bufs[b2],
                                         gsem[b2])

            for b in range(4):
                pltpu.make_async_copy(bufs[b], acc.at[dv.at[CPG - 4 + b]],
                                      ssem[b]).wait()

            if g + 1 < NG:
                pltpu.make_async_copy(src_hbm.at[w, g + 1],
                                      srcv[(g + 1) % 2], isem).wait()
                pltpu.make_async_copy(dst_hbm.at[w, g + 1],
                                      dstv[(g + 1) % 2], isem).wait()

        plsc.subcore_barrier()

        @pl.when(c == 0)
        def _():
            pltpu.sync_copy(acc.at[pl.ds(row0, RPS)],
                            p0_hbm.at[pl.ds(row0, RPS)])

        @pl.when(c == 1)
        def _():
            pltpu.sync_copy(acc.at[pl.ds(row0, RPS)],
                            p1_hbm.at[pl.ds(row0, RPS)])

    return k(h, src2d, dst2d, zeros)


_BN = 2048


def _tc_mlp2(p0, p1, Wa, ba, Wb, bb):
    def body(p0_ref, p1_ref, wa, ba_r, wb, bb_r, o_ref):
        t = p0_ref[...] + p1_ref[...]
        h = jnp.dot(t, wa[...], preferred_element_type=jnp.float32) + ba_r[...]
        h = jnp.maximum(h, 0.0)
        o_ref[...] = jnp.dot(h, wb[...],
                             preferred_element_type=jnp.float32) + bb_r[...]

    return pl.pallas_call(
        body,
        grid=(NP // _BN,),
        in_specs=[
            pl.BlockSpec((_BN, D), lambda i: (i, 0)),
            pl.BlockSpec((_BN, D), lambda i: (i, 0)),
            pl.BlockSpec((D, D), lambda i: (0, 0)),
            pl.BlockSpec((1, D), lambda i: (0, 0)),
            pl.BlockSpec((D, D), lambda i: (0, 0)),
            pl.BlockSpec((1, D), lambda i: (0, 0)),
        ],
        out_specs=pl.BlockSpec((_BN, D), lambda i: (i, 0)),
        out_shape=jax.ShapeDtypeStruct((NP, D), jnp.float32),
    )(p0, p1, Wa, ba.reshape(1, D), Wb, bb.reshape(1, D))


def _tc_linear(p0, p1, W, b):
    def body(p0_ref, p1_ref, w, b_r, o_ref):
        t = p0_ref[...] + p1_ref[...]
        o_ref[...] = jnp.dot(t, w[...],
                             preferred_element_type=jnp.float32) + b_r[...]

    return pl.pallas_call(
        body,
        grid=(NP // _BN,),
        in_specs=[
            pl.BlockSpec((_BN, D), lambda i: (i, 0)),
            pl.BlockSpec((_BN, D), lambda i: (i, 0)),
            pl.BlockSpec((D, D), lambda i: (0, 0)),
            pl.BlockSpec((1, D), lambda i: (0, 0)),
        ],
        out_specs=pl.BlockSpec((_BN, D), lambda i: (i, 0)),
        out_shape=jax.ShapeDtypeStruct((NP, D), jnp.float32),
    )(p0, p1, W, b.reshape(1, D))


def kernel(x, edge_index, W1a, b1a, W1b, b1b, W2a, b2a, W2b, b2b, W3, b3):
    pad = CPW * CH - EPW
    pad_src = jnp.broadcast_to(jnp.arange(pad, dtype=jnp.int32) % N,
                               (NW, pad))
    pad_dst = jnp.broadcast_to(N + jnp.arange(pad, dtype=jnp.int32) % (NP - N),
                               (NW, pad))
    src = jnp.concatenate(
        [edge_index[0].reshape(NW, EPW), pad_src],
        axis=1).reshape(NW, NG, CPG, CH)
    dst = jnp.concatenate(
        [edge_index[1].reshape(NW, EPW), pad_dst],
        axis=1).reshape(NW, NG, CPG, CH)
    zeros = jnp.zeros((NP, D), jnp.float32)
    xp = jnp.pad(x, ((0, NP - N), (0, 0)))

    p0, p1 = _sc_aggregate(xp, src, dst, zeros)
    h = _tc_mlp2(p0, p1, W1a, b1a, W1b, b1b)
    p0, p1 = _sc_aggregate(h, src, dst, zeros)
    h = _tc_mlp2(p0, p1, W2a, b2a, W2b, b2b)
    p0, p1 = _sc_aggregate(h, src, dst, zeros)
    return _tc_linear(p0, p1, W3, b3)[:N]


# speedup vs baseline: 1.0178x; 1.0178x over previous
import functools

import jax
import jax.numpy as jnp
from jax import lax
from jax.experimental import pallas as pl
from jax.experimental.pallas import tpu as pltpu
from jax.experimental.pallas import tpu_sc as plsc

N = 10000
NP = 10240
D = 128
E = 320000
NC = 2
NS = 16
NW = NC * NS
CH = 80
EPW = E // NW
CPW = 128
NG = 4
CPG = CPW // NG
RPS = NP // NS


def _sc_aggregate(h, src2d, dst2d):
    mesh = plsc.VectorSubcoreMesh(core_axis_name="c", subcore_axis_name="s")

    @functools.partial(
        pl.kernel,
        out_type=(jax.ShapeDtypeStruct((NP, D), jnp.float32),
                  jax.ShapeDtypeStruct((NP, D), jnp.float32)),
        mesh=mesh,
        scratch_types=[
            pltpu.VMEM((CPG, CH), jnp.int32),
            pltpu.VMEM((CPG, CH), jnp.int32),
            pltpu.VMEM((CPG, CH), jnp.int32),
            pltpu.VMEM((CPG, CH), jnp.int32),
            pltpu.VMEM((CH, D), jnp.float32),
            pltpu.VMEM((CH, D), jnp.float32),
            pltpu.VMEM_SHARED((NP, D), jnp.float32),
            pltpu.SemaphoreType.DMA,
            pltpu.SemaphoreType.DMA,
            pltpu.SemaphoreType.DMA,
        ],
    )
    def k(h_hbm, src_hbm, dst_hbm, p0_hbm, p1_hbm,
          srcv0, srcv1, dstv0, dstv1, rows0, rows1, acc, sem0, sem1, isem):
        c = lax.axis_index("c")
        s = lax.axis_index("s")
        row0 = s * RPS

        @pl.when(c == 0)
        def _():
            pltpu.sync_copy(h_hbm.at[pl.ds(row0, RPS)],
                            acc.at[pl.ds(row0, RPS)])

        @pl.when(c == 1)
        def _():
            zv = jnp.zeros((16,), jnp.float32)

            @pl.loop(0, CH)
            def _(i):
                @pl.loop(0, D, step=16)
                def _(k2):
                    rows0[i, pl.ds(k2, 16)] = zv

            @pl.loop(0, RPS, step=CH)
            def _(r):
                pltpu.sync_copy(rows0, acc.at[pl.ds(row0 + r, CH)])

        w = c * NS + s
        srcv = (srcv0, srcv1)
        dstv = (dstv0, dstv1)
        pltpu.sync_copy(src_hbm.at[w, 0], srcv0)
        pltpu.sync_copy(dst_hbm.at[w, 0], dstv0)

        plsc.subcore_barrier()

        for g in range(NG):
            sv = srcv[g % 2]
            dv = dstv[g % 2]
            if g + 1 < NG:
                pltpu.async_copy(src_hbm.at[w, g + 1], srcv[(g + 1) % 2], isem)
                pltpu.async_copy(dst_hbm.at[w, g + 1], dstv[(g + 1) % 2], isem)

            pltpu.async_copy(h_hbm.at[sv.at[0]], rows0, sem0)

            @pl.loop(0, CPG, step=2)
            def _(j, sv=sv, dv=dv):
                pltpu.async_copy(h_hbm.at[sv.at[j + 1]], rows1, sem1)
                pltpu.make_async_copy(h_hbm.at[sv.at[j]], rows0, sem0).wait()
                pltpu.sync_copy(rows0, acc.at[dv.at[j]], add=True)

                @pl.when(j + 2 < CPG)
                def _():
                    pltpu.async_copy(h_hbm.at[sv.at[j + 2]], rows0, sem0)

                pltpu.make_async_copy(h_hbm.at[sv.at[j + 1]], rows1,
                                      sem1).wait()
                pltpu.sync_copy(rows1, acc.at[dv.at[j + 1]], add=True)

            if g + 1 < NG:
                pltpu.make_async_copy(src_hbm.at[w, g + 1],
                                      srcv[(g + 1) % 2], isem).wait()
                pltpu.make_async_copy(dst_hbm.at[w, g + 1],
                                      dstv[(g + 1) % 2], isem).wait()

        plsc.subcore_barrier()

        @pl.when(c == 0)
        def _():
            pltpu.sync_copy(acc.at[pl.ds(row0, RPS)],
                            p0_hbm.at[pl.ds(row0, RPS)])

        @pl.when(c == 1)
        def _():
            pltpu.sync_copy(acc.at[pl.ds(row0, RPS)],
                            p1_hbm.at[pl.ds(row0, RPS)])

    return k(h, src2d, dst2d)


_BN = 2048


def _tc_mlp2(p0, p1, Wa, ba, Wb, bb):
    def body(p0_ref, p1_ref, wa, ba_r, wb, bb_r, o_ref):
        t = p0_ref[...] + p1_ref[...]
        h = jnp.dot(t, wa[...], preferred_element_type=jnp.float32) + ba_r[...]
        h = jnp.maximum(h, 0.0)
        o_ref[...] = jnp.dot(h, wb[...],
                             preferred_element_type=jnp.float32) + bb_r[...]

    return pl.pallas_call(
        body,
        grid=(NP // _BN,),
        in_specs=[
            pl.BlockSpec((_BN, D), lambda i: (i, 0)),
            pl.BlockSpec((_BN, D), lambda i: (i, 0)),
            pl.BlockSpec((D, D), lambda i: (0, 0)),
            pl.BlockSpec((1, D), lambda i: (0, 0)),
            pl.BlockSpec((D, D), lambda i: (0, 0)),
            pl.BlockSpec((1, D), lambda i: (0, 0)),
        ],
        out_specs=pl.BlockSpec((_BN, D), lambda i: (i, 0)),
        out_shape=jax.ShapeDtypeStruct((NP, D), jnp.float32),
    )(p0, p1, Wa, ba.reshape(1, D), Wb, bb.reshape(1, D))


def _tc_linear(p0, p1, W, b):
    def body(p0_ref, p1_ref, w, b_r, o_ref):
        t = p0_ref[...] + p1_ref[...]
        o_ref[...] = jnp.dot(t, w[...],
                             preferred_element_type=jnp.float32) + b_r[...]

    return pl.pallas_call(
        body,
        grid=(NP // _BN,),
        in_specs=[
            pl.BlockSpec((_BN, D), lambda i: (i, 0)),
            pl.BlockSpec((_BN, D), lambda i: (i, 0)),
            pl.BlockSpec((D, D), lambda i: (0, 0)),
            pl.BlockSpec((1, D), lambda i: (0, 0)),
        ],
        out_specs=pl.BlockSpec((_BN, D), lambda i: (i, 0)),
        out_shape=jax.ShapeDtypeStruct((NP, D), jnp.float32),
    )(p0, p1, W, b.reshape(1, D))


def kernel(x, edge_index, W1a, b1a, W1b, b1b, W2a, b2a, W2b, b2b, W3, b3):
    pad = CPW * CH - EPW
    pad_src = jnp.broadcast_to(jnp.arange(pad, dtype=jnp.int32) % N,
                               (NW, pad))
    pad_dst = jnp.broadcast_to(N + jnp.arange(pad, dtype=jnp.int32) % (NP - N),
                               (NW, pad))
    src = jnp.concatenate(
        [edge_index[0].reshape(NW, EPW), pad_src],
        axis=1).reshape(NW, NG, CPG, CH)
    dst = jnp.concatenate(
        [edge_index[1].reshape(NW, EPW), pad_dst],
        axis=1).reshape(NW, NG, CPG, CH)
    xp = jnp.pad(x, ((0, NP - N), (0, 0)))

    p0, p1 = _sc_aggregate(xp, src, dst)
    h = _tc_mlp2(p0, p1, W1a, b1a, W1b, b1b)
    p0, p1 = _sc_aggregate(h, src, dst)
    h = _tc_mlp2(p0, p1, W2a, b2a, W2b, b2b)
    p0, p1 = _sc_aggregate(h, src, dst)
    return _tc_linear(p0, p1, W3, b3)[:N]


# speedup vs baseline: 1.0273x; 1.0093x over previous
import functools

import jax
import jax.numpy as jnp
from jax import lax
from jax.experimental import pallas as pl
from jax.experimental.pallas import tpu as pltpu
from jax.experimental.pallas import tpu_sc as plsc

N = 10000
NP = 10240
D = 128
E = 320000
NC = 2
NS = 16
NW = NC * NS
CH = 80
EPW = E // NW
CPW = 128
NG = 4
CPG = CPW // NG
RPS = NP // NS


def _sc_aggregate(h, src2d, dst2d):
    mesh = plsc.VectorSubcoreMesh(core_axis_name="c", subcore_axis_name="s")

    @functools.partial(
        pl.kernel,
        out_type=(jax.ShapeDtypeStruct((NP, D), jnp.float32),
                  jax.ShapeDtypeStruct((NP, D), jnp.float32)),
        mesh=mesh,
        scratch_types=[
            pltpu.VMEM((CPG, CH), jnp.int32),
            pltpu.VMEM((CPG, CH), jnp.int32),
            pltpu.VMEM((CPG, CH), jnp.int32),
            pltpu.VMEM((CPG, CH), jnp.int32),
            pltpu.VMEM((CH, D), jnp.float32),
            pltpu.VMEM((CH, D), jnp.float32),
            pltpu.VMEM_SHARED((NP, D), jnp.float32),
            pltpu.SemaphoreType.DMA,
            pltpu.SemaphoreType.DMA,
            pltpu.SemaphoreType.DMA,
        ],
    )
    def k(h_hbm, src_hbm, dst_hbm, p0_hbm, p1_hbm,
          srcv0, srcv1, dstv0, dstv1, rows0, rows1, acc, sem0, sem1, isem):
        c = lax.axis_index("c")
        s = lax.axis_index("s")
        row0 = s * RPS

        zv = jnp.zeros((16,), jnp.float32)

        @pl.loop(0, CH)
        def _(i):
            @pl.loop(0, D, step=16)
            def _(k2):
                rows0[i, pl.ds(k2, 16)] = zv

        @pl.loop(0, RPS, step=CH)
        def _(r):
            pltpu.sync_copy(rows0, acc.at[pl.ds(row0 + r, CH)])

        w = c * NS + s
        srcv = (srcv0, srcv1)
        dstv = (dstv0, dstv1)
        pltpu.sync_copy(src_hbm.at[w, 0], srcv0)
        pltpu.sync_copy(dst_hbm.at[w, 0], dstv0)

        plsc.subcore_barrier()

        for g in range(NG):
            sv = srcv[g % 2]
            dv = dstv[g % 2]
            if g + 1 < NG:
                pltpu.async_copy(src_hbm.at[w, g + 1], srcv[(g + 1) % 2], isem)
                pltpu.async_copy(dst_hbm.at[w, g + 1], dstv[(g + 1) % 2], isem)

            pltpu.async_copy(h_hbm.at[sv.at[0]], rows0, sem0)

            @pl.loop(0, CPG, step=2)
            def _(j, sv=sv, dv=dv):
                pltpu.async_copy(h_hbm.at[sv.at[j + 1]], rows1, sem1)
                pltpu.make_async_copy(h_hbm.at[sv.at[j]], rows0, sem0).wait()
                pltpu.sync_copy(rows0, acc.at[dv.at[j]], add=True)

                @pl.when(j + 2 < CPG)
                def _():
                    pltpu.async_copy(h_hbm.at[sv.at[j + 2]], rows0, sem0)

                pltpu.make_async_copy(h_hbm.at[sv.at[j + 1]], rows1,
                                      sem1).wait()
                pltpu.sync_copy(rows1, acc.at[dv.at[j + 1]], add=True)

            if g + 1 < NG:
                pltpu.make_async_copy(src_hbm.at[w, g + 1],
                                      srcv[(g + 1) % 2], isem).wait()
                pltpu.make_async_copy(dst_hbm.at[w, g + 1],
                                      dstv[(g + 1) % 2], isem).wait()

        plsc.subcore_barrier()

        @pl.when(c == 0)
        def _():
            pltpu.sync_copy(acc.at[pl.ds(row0, RPS)],
                            p0_hbm.at[pl.ds(row0, RPS)])

        @pl.when(c == 1)
        def _():
            pltpu.sync_copy(acc.at[pl.ds(row0, RPS)],
                            p1_hbm.at[pl.ds(row0, RPS)])

    return k(h, src2d, dst2d)


_BN = 2048


def _tc_mlp2(p0, p1, hp, Wa, ba, Wb, bb):
    def body(p0_ref, p1_ref, h_ref, wa, ba_r, wb, bb_r, o_ref):
        t = p0_ref[...] + p1_ref[...] + h_ref[...]
        h = jnp.dot(t, wa[...], preferred_element_type=jnp.float32) + ba_r[...]
        h = jnp.maximum(h, 0.0)
        o_ref[...] = jnp.dot(h, wb[...],
                             preferred_element_type=jnp.float32) + bb_r[...]

    return pl.pallas_call(
        body,
        grid=(NP // _BN,),
        in_specs=[
            pl.BlockSpec((_BN, D), lambda i: (i, 0)),
            pl.BlockSpec((_BN, D), lambda i: (i, 0)),
            pl.BlockSpec((_BN, D), lambda i: (i, 0)),
            pl.BlockSpec((D, D), lambda i: (0, 0)),
            pl.BlockSpec((1, D), lambda i: (0, 0)),
            pl.BlockSpec((D, D), lambda i: (0, 0)),
            pl.BlockSpec((1, D), lambda i: (0, 0)),
        ],
        out_specs=pl.BlockSpec((_BN, D), lambda i: (i, 0)),
        out_shape=jax.ShapeDtypeStruct((NP, D), jnp.float32),
    )(p0, p1, hp, Wa, ba.reshape(1, D), Wb, bb.reshape(1, D))


def _tc_linear(p0, p1, hp, W, b):
    def body(p0_ref, p1_ref, h_ref, w, b_r, o_ref):
        t = p0_ref[...] + p1_ref[...] + h_ref[...]
        o_ref[...] = jnp.dot(t, w[...],
                             preferred_element_type=jnp.float32) + b_r[...]

    return pl.pallas_call(
        body,
        grid=(NP // _BN,),
        in_specs=[
            pl.BlockSpec((_BN, D), lambda i: (i, 0)),
            pl.BlockSpec((_BN, D), lambda i: (i, 0)),
            pl.BlockSpec((_BN, D), lambda i: (i, 0)),
            pl.BlockSpec((D, D), lambda i: (0, 0)),
            pl.BlockSpec((1, D), lambda i: (0, 0)),
        ],
        out_specs=pl.BlockSpec((_BN, D), lambda i: (i, 0)),
        out_shape=jax.ShapeDtypeStruct((NP, D), jnp.float32),
    )(p0, p1, hp, W, b.reshape(1, D))


def kernel(x, edge_index, W1a, b1a, W1b, b1b, W2a, b2a, W2b, b2b, W3, b3):
    pad = CPW * CH - EPW
    pad_src = jnp.broadcast_to(jnp.arange(pad, dtype=jnp.int32) % N,
                               (NW, pad))
    pad_dst = jnp.broadcast_to(N + jnp.arange(pad, dtype=jnp.int32) % (NP - N),
                               (NW, pad))
    src = jnp.concatenate(
        [edge_index[0].reshape(NW, EPW), pad_src],
        axis=1).reshape(NW, NG, CPG, CH)
    dst = jnp.concatenate(
        [edge_index[1].reshape(NW, EPW), pad_dst],
        axis=1).reshape(NW, NG, CPG, CH)
    xp = jnp.pad(x, ((0, NP - N), (0, 0)))

    p0, p1 = _sc_aggregate(xp, src, dst)
    h = _tc_mlp2(p0, p1, xp, W1a, b1a, W1b, b1b)
    p0, p1 = _sc_aggregate(h, src, dst)
    h = _tc_mlp2(p0, p1, h, W2a, b2a, W2b, b2b)
    p0, p1 = _sc_aggregate(h, src, dst)
    return _tc_linear(p0, p1, h, W3, b3)[:N]
